# baseline (speedup 1.0000x reference)
import functools

import jax
import jax.numpy as jnp
from jax import lax
from jax.experimental import pallas as pl
from jax.experimental.pallas import tpu as pltpu
from jax.experimental.pallas import tpu_sc as plsc

N = 16384
ARITY = 2
D_C = 32
D_P = 32
D_ATOM = 64
NP = 100
NUM_CONST = 1000000

NC = 2
NS = 16
L = 16
NW = NC * NS
TPW = N // NW
IDX_CHUNK = 128
CT_CH = ARITY * TPW // IDX_CHUNK
GROUP = 128 // D_C


TBLK = 65536
TGRID = -(-NUM_CONST // TBLK)
CROWS = TGRID * (TBLK // GROUP)


def _relayout_body(tt_ref, out_ref):
    for a in range(TBLK // 512):
        s = jnp.concatenate(
            [tt_ref[:, 128 * (4 * a + q):128 * (4 * a + q + 1)]
             for q in range(GROUP)], axis=0)
        out_ref[pl.ds(128 * a, 128), :] = jnp.transpose(s)


def _tc_relayout(tableT):
    return pl.pallas_call(
        _relayout_body,
        grid=(TGRID,),
        in_specs=[pl.BlockSpec((D_C, TBLK), lambda j: (0, j))],
        out_specs=pl.BlockSpec((TBLK // GROUP, 128), lambda j: (j, 0)),
        out_shape=jax.ShapeDtypeStruct((CROWS, 128), jnp.float32),
    )(tableT)


_SC_MESH = plsc.VectorSubcoreMesh(
    core_axis_name="c", subcore_axis_name="s", num_cores=NC, num_subcores=NS
)
_SC_PARAMS = pltpu.CompilerParams(
    needs_layout_passes=False, use_tc_tiling_on_sc=True)


def _sc_compose(x_domain, triT):
    @functools.partial(
        pl.kernel,
        out_type=(jax.ShapeDtypeStruct((NW * CT_CH, IDX_CHUNK), jnp.int32),
                  jax.ShapeDtypeStruct((NW * CT_CH, IDX_CHUNK), jnp.int32)),
        mesh=_SC_MESH,
        compiler_params=_SC_PARAMS,
        scratch_types=[
            pltpu.VMEM((N,), jnp.int32),
            pltpu.VMEM((ARITY, TPW), jnp.int32),
            pltpu.VMEM((CT_CH, IDX_CHUNK), jnp.int32),
            pltpu.VMEM((CT_CH, IDX_CHUNK), jnp.int32),
        ],
    )
    def k(xd_hbm, tri_hbm, grp_out, soff_out, xdom_v, tri_v, grp_v, soff_v):
        wid = lax.axis_index("s") * NC + lax.axis_index("c")
        pltpu.sync_copy(xd_hbm, xdom_v)
        pltpu.sync_copy(tri_hbm.at[:, pl.ds(wid * TPW, TPW)], tri_v)
        for s in range(ARITY):
            for i in range(TPW // L):
                iv = tri_v[s, pl.ds(i * L, L)]
                idx2 = plsc.load_gather(xdom_v, [iv])
                grp = lax.bitwise_or(
                    lax.shift_left(lax.shift_right_logical(idx2, 9), 7),
                    lax.bitwise_and(idx2, 127))
                soff = lax.shift_left(
                    lax.bitwise_and(lax.shift_right_logical(idx2, 7), 3), 5)
                row = jnp.zeros((L,), jnp.int32) + ((32 * i) // 128)
                col = (lax.iota(jnp.int32, L) * 2
                       + (32 * (i % 4) + s))
                plsc.store_scatter(grp_v, [row, col], grp)
                plsc.store_scatter(soff_v, [row, col], soff)
        pltpu.sync_copy(grp_v, grp_out.at[pl.ds(wid * CT_CH, CT_CH)])
        pltpu.sync_copy(soff_v, soff_out.at[pl.ds(wid * CT_CH, CT_CH)])

    return k(x_domain, triT)


def _sc_gather(grp2d, soff2d, table4):
    @functools.partial(
        pl.kernel,
        out_type=jax.ShapeDtypeStruct((ARITY * N * D_C // 128, 128),
                                      jnp.float32),
        mesh=_SC_MESH,
        compiler_params=_SC_PARAMS,
        scratch_types=[
            pltpu.VMEM((CT_CH, IDX_CHUNK), jnp.int32),
            pltpu.VMEM((CT_CH, IDX_CHUNK), jnp.int32),
            pltpu.VMEM((4, IDX_CHUNK, 128), jnp.float32),
            pltpu.VMEM((ARITY * TPW * D_C // 128, 128), jnp.float32),
            pltpu.SemaphoreType.DMA,
            pltpu.SemaphoreType.DMA,
            pltpu.SemaphoreType.DMA,
            pltpu.SemaphoreType.DMA,
        ],
    )
    def k(grp_hbm, soff_hbm, tab4_hbm, ct_out,
          grp_v, soff_v, gbuf_v, ct_v, sem0, sem1, sem2, sem3):
        wid = lax.axis_index("s") * NC + lax.axis_index("c")
        pltpu.sync_copy(grp_hbm.at[pl.ds(wid * CT_CH, CT_CH)], grp_v)
        pltpu.sync_copy(soff_hbm.at[pl.ds(wid * CT_CH, CT_CH)], soff_v)

        DEPTH = 4
        sems = (sem0, sem1, sem2, sem3)
        cps = [None] * DEPTH
        for j in range(DEPTH - 1):
            cps[j] = pltpu.async_copy(
                tab4_hbm.at[grp_v.at[j]], gbuf_v.at[j], sems[j])
        for j in range(CT_CH):
            jn = j + DEPTH - 1
            if jn < CT_CH:
                cps[jn % DEPTH] = pltpu.async_copy(
                    tab4_hbm.at[grp_v.at[jn]], gbuf_v.at[jn % DEPTH],
                    sems[jn % DEPTH])
            cps[j % DEPTH].wait()
            buf = gbuf_v.at[j % DEPTH]
            for g in range(IDX_CHUNK // L):
                rows = lax.iota(jnp.int32, L) + (g * L)
                base = (rows + j * IDX_CHUNK) * D_C
                soff = soff_v[j, pl.ds(g * L, L)]

                @plsc.parallel_loop(0, D_C, unroll=8)
                def _ext(c, buf=buf, rows=rows, base=base, soff=soff):
                    vals = plsc.load_gather(buf, [rows, soff + c])
                    flat = base + c
                    plsc.store_scatter(
                        ct_v,
                        [lax.shift_right_logical(flat, 7),
                         lax.bitwise_and(flat, 127)],
                        vals)

        prows = ARITY * TPW * D_C // 128
        pltpu.sync_copy(ct_v, ct_out.at[pl.ds(prows * wid, prows)])

    return k(grp2d, soff2d, table4)


BN = 2048


def _mlp_body(pid_ref, ct_ref, ptab_ref, w1p_ref, w1c_ref, b1_ref, w2_ref,
              b2_ref, emb_ref, out_ref):
    pt1 = jnp.dot(ptab_ref[...], w1p_ref[...],
                  preferred_element_type=jnp.float32)
    oh = (pid_ref[...] == lax.broadcasted_iota(jnp.int32, (1, NP), 1))
    h = jnp.dot(oh.astype(jnp.float32), pt1,
                preferred_element_type=jnp.float32)
    pk = ct_ref[...]
    ha = jnp.dot(pk[:, :ARITY * D_C], w1c_ref[...],
                 preferred_element_type=jnp.float32)
    hb = jnp.dot(pk[:, ARITY * D_C:], w1c_ref[...],
                 preferred_element_type=jnp.float32)
    hc = jnp.concatenate([ha[:, None, :], hb[:, None, :]], axis=1)
    h = h + hc.reshape(h.shape)
    h = jnp.maximum(h + b1_ref[...], 0.0)
    emb_ref[...] = jnp.transpose(h)
    o = jnp.dot(h, w2_ref[...], preferred_element_type=jnp.float32)
    out_ref[...] = jax.nn.sigmoid(o + b2_ref[...])


def _tc_mlp(pid, ct, ptab, W1, b1, W2, b2):
    w1p = W1[:D_P]
    w1c = W1[D_P:]
    emb, out = pl.pallas_call(
        _mlp_body,
        grid=(N // BN,),
        in_specs=[
            pl.BlockSpec((BN, 1), lambda i: (i, 0)),
            pl.BlockSpec((BN // 2, 128), lambda i: (i, 0)),
            pl.BlockSpec((NP, D_P), lambda i: (0, 0)),
            pl.BlockSpec((D_P, D_ATOM), lambda i: (0, 0)),
            pl.BlockSpec((ARITY * D_C, D_ATOM), lambda i: (0, 0)),
            pl.BlockSpec((1, D_ATOM), lambda i: (0, 0)),
            pl.BlockSpec((D_ATOM, 1), lambda i: (0, 0)),
            pl.BlockSpec((1, 1), lambda i: (0, 0)),
        ],
        out_specs=[
            pl.BlockSpec((D_ATOM, BN), lambda i: (0, i)),
            pl.BlockSpec((BN, 1), lambda i: (i, 0)),
        ],
        out_shape=[
            jax.ShapeDtypeStruct((D_ATOM, N), jnp.float32),
            jax.ShapeDtypeStruct((N, 1), jnp.float32),
        ],
    )(pid, ct, ptab, w1p, w1c, b1.reshape(1, D_ATOM), W2, b2.reshape(1, 1))
    return emb.T, out


def kernel(X_domain, triplet_idx, pred_ids, constant_table, predicate_table,
           W1, b1, W2, b2):
    grp2d, soff2d = _sc_compose(X_domain.astype(jnp.int32),
                                triplet_idx.astype(jnp.int32).T)
    table4 = _tc_relayout(constant_table.T)
    ct_packed = _sc_gather(grp2d, soff2d, table4)
    emb, out = _tc_mlp(pred_ids.astype(jnp.int32).reshape(N, 1), ct_packed,
                       predicate_table, W1, b1, W2, b2)
    return out.reshape(N, 1, 1), emb


# speedup vs baseline: 1.0154x; 1.0154x over previous
import functools

import jax
import jax.numpy as jnp
from jax import lax
from jax.experimental import pallas as pl
from jax.experimental.pallas import tpu as pltpu
from jax.experimental.pallas import tpu_sc as plsc

N = 16384
ARITY = 2
D_C = 32
D_P = 32
D_ATOM = 64
NP = 100
NUM_CONST = 1000000

NC = 2
NS = 16
L = 16
NW = NC * NS
TPW = N // NW
IDX_CHUNK = 128
CT_CH = ARITY * TPW // IDX_CHUNK
GROUP = 128 // D_C


TBLK = 65536
TGRID = -(-NUM_CONST // TBLK)
CROWS = TGRID * (TBLK // GROUP)


def _relayout_body(tt_ref, out_ref):
    for a in range(TBLK // 512):
        s = jnp.concatenate(
            [tt_ref[:, 128 * (4 * a + q):128 * (4 * a + q + 1)]
             for q in range(GROUP)], axis=0)
        out_ref[pl.ds(128 * a, 128), :] = jnp.transpose(s)


def _tc_relayout(tableT):
    return pl.pallas_call(
        _relayout_body,
        grid=(TGRID,),
        in_specs=[pl.BlockSpec((D_C, TBLK), lambda j: (0, j))],
        out_specs=pl.BlockSpec((TBLK // GROUP, 128), lambda j: (j, 0)),
        out_shape=jax.ShapeDtypeStruct((CROWS, 128), jnp.float32),
    )(tableT)


_SC_MESH = plsc.VectorSubcoreMesh(
    core_axis_name="c", subcore_axis_name="s", num_cores=NC, num_subcores=NS
)
_SC_PARAMS = pltpu.CompilerParams(
    needs_layout_passes=False, use_tc_tiling_on_sc=True)


def _sc_compose(x_domain, triT):
    @functools.partial(
        pl.kernel,
        out_type=(jax.ShapeDtypeStruct((NW * CT_CH, IDX_CHUNK), jnp.int32),
                  jax.ShapeDtypeStruct((NW * CT_CH, IDX_CHUNK), jnp.int32)),
        mesh=_SC_MESH,
        compiler_params=_SC_PARAMS,
        scratch_types=[
            pltpu.VMEM((N,), jnp.int32),
            pltpu.VMEM((ARITY, TPW), jnp.int32),
            pltpu.VMEM((CT_CH, IDX_CHUNK), jnp.int32),
            pltpu.VMEM((CT_CH, IDX_CHUNK), jnp.int32),
        ],
    )
    def k(xd_hbm, tri_hbm, grp_out, soff_out, xdom_v, tri_v, grp_v, soff_v):
        wid = lax.axis_index("s") * NC + lax.axis_index("c")
        pltpu.sync_copy(xd_hbm, xdom_v)
        pltpu.sync_copy(tri_hbm.at[:, pl.ds(wid * TPW, TPW)], tri_v)
        for s in range(ARITY):
            for i in range(TPW // L):
                iv = tri_v[s, pl.ds(i * L, L)]
                idx2 = plsc.load_gather(xdom_v, [iv])
                grp = lax.bitwise_or(
                    lax.shift_left(lax.shift_right_logical(idx2, 9), 7),
                    lax.bitwise_and(idx2, 127))
                soff = lax.shift_left(
                    lax.bitwise_and(lax.shift_right_logical(idx2, 7), 3), 5)
                row = jnp.zeros((L,), jnp.int32) + ((32 * i) // 128)
                col = (lax.iota(jnp.int32, L) * 2
                       + (32 * (i % 4) + s))
                plsc.store_scatter(grp_v, [row, col], grp)
                plsc.store_scatter(soff_v, [row, col], soff)
        pltpu.sync_copy(grp_v, grp_out.at[pl.ds(wid * CT_CH, CT_CH)])
        pltpu.sync_copy(soff_v, soff_out.at[pl.ds(wid * CT_CH, CT_CH)])

    return k(x_domain, triT)


def _sc_gather(grp2d, soff2d, table4):
    @functools.partial(
        pl.kernel,
        out_type=jax.ShapeDtypeStruct((ARITY * N * D_C // 128, 128),
                                      jnp.float32),
        mesh=_SC_MESH,
        compiler_params=_SC_PARAMS,
        scratch_types=[
            pltpu.VMEM((CT_CH, IDX_CHUNK), jnp.int32),
            pltpu.VMEM((CT_CH, IDX_CHUNK), jnp.int32),
            pltpu.VMEM((4, IDX_CHUNK, 128), jnp.float32),
            pltpu.VMEM((ARITY * TPW * D_C // 128, 128), jnp.float32),
            pltpu.SemaphoreType.DMA,
            pltpu.SemaphoreType.DMA,
            pltpu.SemaphoreType.DMA,
            pltpu.SemaphoreType.DMA,
        ],
    )
    def k(grp_hbm, soff_hbm, tab4_hbm, ct_out,
          grp_v, soff_v, gbuf_v, ct_v, sem0, sem1, sem2, sem3):
        wid = lax.axis_index("s") * NC + lax.axis_index("c")
        pltpu.sync_copy(grp_hbm.at[pl.ds(wid * CT_CH, CT_CH)], grp_v)
        pltpu.sync_copy(soff_hbm.at[pl.ds(wid * CT_CH, CT_CH)], soff_v)

        DEPTH = 2
        sems = (sem0, sem1, sem2, sem3)
        cps = [None] * DEPTH
        for j in range(DEPTH - 1):
            cps[j] = pltpu.async_copy(
                tab4_hbm.at[grp_v.at[j]], gbuf_v.at[j], sems[j])
        for j in range(CT_CH):
            jn = j + DEPTH - 1
            if jn < CT_CH:
                cps[jn % DEPTH] = pltpu.async_copy(
                    tab4_hbm.at[grp_v.at[jn]], gbuf_v.at[jn % DEPTH],
                    sems[jn % DEPTH])
            cps[j % DEPTH].wait()
            buf = gbuf_v.at[j % DEPTH]
            for g in range(IDX_CHUNK // L):
                rows = lax.iota(jnp.int32, L) + (g * L)
                base = (rows + j * IDX_CHUNK) * D_C
                soff = soff_v[j, pl.ds(g * L, L)]

                @plsc.parallel_loop(0, D_C, unroll=8)
                def _ext(c, buf=buf, rows=rows, base=base, soff=soff):
                    vals = plsc.load_gather(buf, [rows, soff + c])
                    flat = base + c
                    plsc.store_scatter(
                        ct_v,
                        [lax.shift_right_logical(flat, 7),
                         lax.bitwise_and(flat, 127)],
                        vals)

        prows = ARITY * TPW * D_C // 128
        pltpu.sync_copy(ct_v, ct_out.at[pl.ds(prows * wid, prows)])

    return k(grp2d, soff2d, table4)


BN = 4096


def _mlp_body(pid_ref, ct_ref, ptab_ref, w1p_ref, w1c_ref, b1_ref, w2_ref,
              b2_ref, emb_ref, out_ref):
    pt1 = jnp.dot(ptab_ref[...], w1p_ref[...],
                  preferred_element_type=jnp.float32)
    oh = (pid_ref[...] == lax.broadcasted_iota(jnp.int32, (1, NP), 1))
    h = jnp.dot(oh.astype(jnp.float32), pt1,
                preferred_element_type=jnp.float32)
    pk = ct_ref[...]
    ha = jnp.dot(pk[:, :ARITY * D_C], w1c_ref[...],
                 preferred_element_type=jnp.float32)
    hb = jnp.dot(pk[:, ARITY * D_C:], w1c_ref[...],
                 preferred_element_type=jnp.float32)
    hc = jnp.concatenate([ha[:, None, :], hb[:, None, :]], axis=1)
    h = h + hc.reshape(h.shape)
    h = jnp.maximum(h + b1_ref[...], 0.0)
    emb_ref[...] = jnp.transpose(h)
    o = jnp.dot(h, w2_ref[...], preferred_element_type=jnp.float32)
    out_ref[...] = jax.nn.sigmoid(o + b2_ref[...])


def _tc_mlp(pid, ct, ptab, W1, b1, W2, b2):
    w1p = W1[:D_P]
    w1c = W1[D_P:]
    emb, out = pl.pallas_call(
        _mlp_body,
        grid=(N // BN,),
        in_specs=[
            pl.BlockSpec((BN, 1), lambda i: (i, 0)),
            pl.BlockSpec((BN // 2, 128), lambda i: (i, 0)),
            pl.BlockSpec((NP, D_P), lambda i: (0, 0)),
            pl.BlockSpec((D_P, D_ATOM), lambda i: (0, 0)),
            pl.BlockSpec((ARITY * D_C, D_ATOM), lambda i: (0, 0)),
            pl.BlockSpec((1, D_ATOM), lambda i: (0, 0)),
            pl.BlockSpec((D_ATOM, 1), lambda i: (0, 0)),
            pl.BlockSpec((1, 1), lambda i: (0, 0)),
        ],
        out_specs=[
            pl.BlockSpec((D_ATOM, BN), lambda i: (0, i)),
            pl.BlockSpec((BN, 1), lambda i: (i, 0)),
        ],
        out_shape=[
            jax.ShapeDtypeStruct((D_ATOM, N), jnp.float32),
            jax.ShapeDtypeStruct((N, 1), jnp.float32),
        ],
    )(pid, ct, ptab, w1p, w1c, b1.reshape(1, D_ATOM), W2, b2.reshape(1, 1))
    return emb.T, out


def kernel(X_domain, triplet_idx, pred_ids, constant_table, predicate_table,
           W1, b1, W2, b2):
    grp2d, soff2d = _sc_compose(X_domain.astype(jnp.int32),
                                triplet_idx.astype(jnp.int32).T)
    table4 = _tc_relayout(constant_table.T)
    ct_packed = _sc_gather(grp2d, soff2d, table4)
    emb, out = _tc_mlp(pred_ids.astype(jnp.int32).reshape(N, 1), ct_packed,
                       predicate_table, W1, b1, W2, b2)
    return out.reshape(N, 1, 1), emb


# speedup vs baseline: 1.0224x; 1.0069x over previous
import functools

import jax
import jax.numpy as jnp
from jax import lax
from jax.experimental import pallas as pl
from jax.experimental.pallas import tpu as pltpu
from jax.experimental.pallas import tpu_sc as plsc

N = 16384
ARITY = 2
D_C = 32
D_P = 32
D_ATOM = 64
NP = 100
NUM_CONST = 1000000

NC = 2
NS = 16
L = 16
NW = NC * NS
TPW = N // NW
IDX_CHUNK = 128
CT_CH = ARITY * TPW // IDX_CHUNK
GROUP = 128 // D_C


TBLK = 65536
TGRID = -(-NUM_CONST // TBLK)
CROWS = TGRID * (TBLK // GROUP)


def _relayout_body(tt_ref, out_ref):
    for a in range(TBLK // 512):
        s = jnp.concatenate(
            [tt_ref[:, 128 * (4 * a + q):128 * (4 * a + q + 1)]
             for q in range(GROUP)], axis=0)
        out_ref[pl.ds(128 * a, 128), :] = jnp.transpose(s)


def _tc_relayout(tableT):
    return pl.pallas_call(
        _relayout_body,
        grid=(TGRID,),
        in_specs=[pl.BlockSpec((D_C, TBLK), lambda j: (0, j))],
        out_specs=pl.BlockSpec((TBLK // GROUP, 128), lambda j: (j, 0)),
        out_shape=jax.ShapeDtypeStruct((CROWS, 128), jnp.float32),
    )(tableT)


_SC_MESH = plsc.VectorSubcoreMesh(
    core_axis_name="c", subcore_axis_name="s", num_cores=NC, num_subcores=NS
)
_SC_PARAMS = pltpu.CompilerParams(
    needs_layout_passes=False, use_tc_tiling_on_sc=True)


def _sc_compose(x_domain, triT):
    @functools.partial(
        pl.kernel,
        out_type=(jax.ShapeDtypeStruct((NW * CT_CH, IDX_CHUNK), jnp.int32),
                  jax.ShapeDtypeStruct((NW * CT_CH, IDX_CHUNK), jnp.int32)),
        mesh=_SC_MESH,
        compiler_params=_SC_PARAMS,
        scratch_types=[
            pltpu.VMEM((N,), jnp.int32),
            pltpu.VMEM((ARITY, TPW), jnp.int32),
            pltpu.VMEM((CT_CH, IDX_CHUNK), jnp.int32),
            pltpu.VMEM((CT_CH, IDX_CHUNK), jnp.int32),
        ],
    )
    def k(xd_hbm, tri_hbm, grp_out, soff_out, xdom_v, tri_v, grp_v, soff_v):
        wid = lax.axis_index("s") * NC + lax.axis_index("c")
        pltpu.sync_copy(xd_hbm, xdom_v)
        pltpu.sync_copy(tri_hbm.at[:, pl.ds(wid * TPW, TPW)], tri_v)
        for s in range(ARITY):
            for i in range(TPW // L):
                iv = tri_v[s, pl.ds(i * L, L)]
                idx2 = plsc.load_gather(xdom_v, [iv])
                grp = lax.bitwise_or(
                    lax.shift_left(lax.shift_right_logical(idx2, 9), 7),
                    lax.bitwise_and(idx2, 127))
                soff = lax.shift_left(
                    lax.bitwise_and(lax.shift_right_logical(idx2, 7), 3), 5)
                row = jnp.zeros((L,), jnp.int32) + ((32 * i) // 128)
                col = (lax.iota(jnp.int32, L) * 2
                       + (32 * (i % 4) + s))
                plsc.store_scatter(grp_v, [row, col], grp)
                plsc.store_scatter(soff_v, [row, col], soff)
        pltpu.sync_copy(grp_v, grp_out.at[pl.ds(wid * CT_CH, CT_CH)])
        pltpu.sync_copy(soff_v, soff_out.at[pl.ds(wid * CT_CH, CT_CH)])

    return k(x_domain, triT)


def _sc_gather(grp2d, soff2d, table4):
    @functools.partial(
        pl.kernel,
        out_type=jax.ShapeDtypeStruct((ARITY * N * D_C // 128, 128),
                                      jnp.float32),
        mesh=_SC_MESH,
        compiler_params=_SC_PARAMS,
        scratch_types=[
            pltpu.VMEM((CT_CH, IDX_CHUNK), jnp.int32),
            pltpu.VMEM((CT_CH, IDX_CHUNK), jnp.int32),
            pltpu.VMEM((4, IDX_CHUNK, 128), jnp.float32),
            pltpu.VMEM((ARITY * TPW * D_C // 128, 128), jnp.float32),
            pltpu.SemaphoreType.DMA,
            pltpu.SemaphoreType.DMA,
            pltpu.SemaphoreType.DMA,
            pltpu.SemaphoreType.DMA,
            pltpu.SemaphoreType.DMA,
        ],
    )
    def k(grp_hbm, soff_hbm, tab4_hbm, ct_out,
          grp_v, soff_v, gbuf_v, ct_v, sem0, sem1, sem2, sem3, semw):
        wid = lax.axis_index("s") * NC + lax.axis_index("c")
        pltpu.sync_copy(grp_hbm.at[pl.ds(wid * CT_CH, CT_CH)], grp_v)
        pltpu.sync_copy(soff_hbm.at[pl.ds(wid * CT_CH, CT_CH)], soff_v)

        DEPTH = 2
        sems = (sem0, sem1, sem2, sem3)
        cps = [None] * DEPTH
        wcps = []
        for j in range(DEPTH - 1):
            cps[j] = pltpu.async_copy(
                tab4_hbm.at[grp_v.at[j]], gbuf_v.at[j], sems[j])
        for j in range(CT_CH):
            jn = j + DEPTH - 1
            if jn < CT_CH:
                cps[jn % DEPTH] = pltpu.async_copy(
                    tab4_hbm.at[grp_v.at[jn]], gbuf_v.at[jn % DEPTH],
                    sems[jn % DEPTH])
            cps[j % DEPTH].wait()
            buf = gbuf_v.at[j % DEPTH]
            for g in range(IDX_CHUNK // L):
                rows = lax.iota(jnp.int32, L) + (g * L)
                base = (rows + j * IDX_CHUNK) * D_C
                soff = soff_v[j, pl.ds(g * L, L)]

                @plsc.parallel_loop(0, D_C, unroll=8)
                def _ext(c, buf=buf, rows=rows, base=base, soff=soff):
                    vals = plsc.load_gather(buf, [rows, soff + c])
                    flat = base + c
                    plsc.store_scatter(
                        ct_v,
                        [lax.shift_right_logical(flat, 7),
                         lax.bitwise_and(flat, 127)],
                        vals)

            pr_ch = IDX_CHUNK * D_C // 128
            wcps.append(pltpu.async_copy(
                ct_v.at[pl.ds(j * pr_ch, pr_ch)],
                ct_out.at[pl.ds((CT_CH * wid + j) * pr_ch, pr_ch)], semw))
        for c in wcps:
            c.wait()

    return k(grp2d, soff2d, table4)


BN = 4096


def _mlp_body(pid_ref, ct_ref, ptab_ref, w1p_ref, w1c_ref, b1_ref, w2_ref,
              b2_ref, emb_ref, out_ref):
    pt1 = jnp.dot(ptab_ref[...], w1p_ref[...],
                  preferred_element_type=jnp.float32)
    oh = (pid_ref[...] == lax.broadcasted_iota(jnp.int32, (1, NP), 1))
    h = jnp.dot(oh.astype(jnp.float32), pt1,
                preferred_element_type=jnp.float32)
    pk = ct_ref[...]
    ha = jnp.dot(pk[:, :ARITY * D_C], w1c_ref[...],
                 preferred_element_type=jnp.float32)
    hb = jnp.dot(pk[:, ARITY * D_C:], w1c_ref[...],
                 preferred_element_type=jnp.float32)
    hc = jnp.concatenate([ha[:, None, :], hb[:, None, :]], axis=1)
    h = h + hc.reshape(h.shape)
    h = jnp.maximum(h + b1_ref[...], 0.0)
    emb_ref[...] = jnp.transpose(h)
    o = jnp.dot(h, w2_ref[...], preferred_element_type=jnp.float32)
    out_ref[...] = jax.nn.sigmoid(o + b2_ref[...])


def _tc_mlp(pid, ct, ptab, W1, b1, W2, b2):
    w1p = W1[:D_P]
    w1c = W1[D_P:]
    emb, out = pl.pallas_call(
        _mlp_body,
        grid=(N // BN,),
        in_specs=[
            pl.BlockSpec((BN, 1), lambda i: (i, 0)),
            pl.BlockSpec((BN // 2, 128), lambda i: (i, 0)),
            pl.BlockSpec((NP, D_P), lambda i: (0, 0)),
            pl.BlockSpec((D_P, D_ATOM), lambda i: (0, 0)),
            pl.BlockSpec((ARITY * D_C, D_ATOM), lambda i: (0, 0)),
            pl.BlockSpec((1, D_ATOM), lambda i: (0, 0)),
            pl.BlockSpec((D_ATOM, 1), lambda i: (0, 0)),
            pl.BlockSpec((1, 1), lambda i: (0, 0)),
        ],
        out_specs=[
            pl.BlockSpec((D_ATOM, BN), lambda i: (0, i)),
            pl.BlockSpec((BN, 1), lambda i: (i, 0)),
        ],
        out_shape=[
            jax.ShapeDtypeStruct((D_ATOM, N), jnp.float32),
            jax.ShapeDtypeStruct((N, 1), jnp.float32),
        ],
    )(pid, ct, ptab, w1p, w1c, b1.reshape(1, D_ATOM), W2, b2.reshape(1, 1))
    return emb.T, out


def kernel(X_domain, triplet_idx, pred_ids, constant_table, predicate_table,
           W1, b1, W2, b2):
    grp2d, soff2d = _sc_compose(X_domain.astype(jnp.int32),
                                triplet_idx.astype(jnp.int32).T)
    table4 = _tc_relayout(constant_table.T)
    ct_packed = _sc_gather(grp2d, soff2d, table4)
    emb, out = _tc_mlp(pred_ids.astype(jnp.int32).reshape(N, 1), ct_packed,
                       predicate_table, W1, b1, W2, b2)
    return out.reshape(N, 1, 1), emb
